# Initial kernel scaffold; baseline (speedup 1.0000x reference)
import jax
import jax.numpy as jnp
from jax.experimental import pallas as pl


def kernel(h, edge_m, norm, edge_index, W, b, ln_g, ln_b):
    raise NotImplementedError("write your pallas kernel here")


# speedup vs baseline: 2.9796x; 2.9796x over previous
import functools

import jax
import jax.numpy as jnp
from jax import lax
from jax.experimental import pallas as pl
from jax.experimental.pallas import tpu as pltpu
from jax.experimental.pallas import tpu_sc as plsc

N_NODES = 100000
N_EDGES = 3200000
F_EDGE = 24
IN_FEATS = 128
OUT_FEATS = 128

NC = 2
NS = 16
HALF = N_NODES // NC
ACC_ROWS = 50048
ZERO_PER_TILE = ACC_ROWS // NS

E_PER_TILE = N_EDGES // NS
CHUNK = 2000
N_CHUNKS = E_PER_TILE // CHUNK
MB = 80
NMB = CHUNK // MB


def _segsum_body(em_hbm, dst_hbm, out_hbm, idx_buf, row_buf, lidx_buf, acc):
    c = lax.axis_index("c")
    s = lax.axis_index("s")
    if True:
        base = c * HALF

        zeros16 = jnp.zeros((16,), jnp.float32)

        def zrow(i, _):
            row_buf[i, pl.ds(0, 16)] = zeros16
            row_buf[i, pl.ds(8, 16)] = zeros16
            return 0

        lax.fori_loop(0, CHUNK, zrow, 0)
        z0 = s * ZERO_PER_TILE
        pltpu.sync_copy(row_buf, acc.at[pl.ds(z0, CHUNK), :])
        pltpu.sync_copy(row_buf.at[pl.ds(0, ZERO_PER_TILE - CHUNK), :],
                        acc.at[pl.ds(z0 + CHUNK, ZERO_PER_TILE - CHUNK), :])
        plsc.subcore_barrier()

        tile_base = s * E_PER_TILE

        def chunk_body(k, _):
            start = tile_base + k * CHUNK
            pltpu.sync_copy(dst_hbm.at[pl.ds(start, CHUNK)], idx_buf)
            pltpu.sync_copy(em_hbm.at[pl.ds(start, CHUNK), :], row_buf)

            def remap(g, _):
                j = g // (MB // 16)
                i = g % (MB // 16)
                v = idx_buf[pl.ds(g * 16, 16)]
                li = v - base
                ok = (li >= 0) & (li < HALF)
                lidx_buf[j, pl.ds(i * 16, 16)] = jnp.where(ok, li, HALF)
                return 0

            lax.fori_loop(0, CHUNK // 16, remap, 0)

            def scat(j, _):
                pltpu.sync_copy(row_buf.at[pl.ds(j * MB, MB), :],
                                acc.at[lidx_buf.at[j]], add=True)
                return 0

            lax.fori_loop(0, NMB, scat, 0)
            return 0

        lax.fori_loop(0, N_CHUNKS, chunk_body, 0)
        plsc.subcore_barrier()

        @pl.when(s < NS - 1)
        def _():
            r0 = s * 3200
            pltpu.sync_copy(acc.at[pl.ds(r0, 3200), :],
                            out_hbm.at[pl.ds(c * HALF + r0, 3200), :])

        @pl.when(s == NS - 1)
        def _():
            r0 = (NS - 1) * 3200
            pltpu.sync_copy(acc.at[pl.ds(r0, 2000), :],
                            out_hbm.at[pl.ds(c * HALF + r0, 2000), :])


@functools.partial(
    pl.kernel,
    out_type=jax.ShapeDtypeStruct((N_NODES, F_EDGE), jnp.float32),
    mesh=plsc.VectorSubcoreMesh(core_axis_name="c", subcore_axis_name="s",
                                num_cores=NC, num_subcores=NS),
    scratch_types=[
        pltpu.VMEM((CHUNK,), jnp.int32),
        pltpu.VMEM((CHUNK, F_EDGE), jnp.float32),
        pltpu.VMEM((NMB, MB), jnp.int32),
        pltpu.VMEM_SHARED((ACC_ROWS, F_EDGE), jnp.float32),
    ],
    compiler_params=pltpu.CompilerParams(use_tc_tiling_on_sc=False),
)
def _segment_sum_sc(em_hbm, dst_hbm, out_hbm, idx_buf, row_buf, lidx_buf, acc):
    _segsum_body(em_hbm, dst_hbm, out_hbm, idx_buf, row_buf, lidx_buf, acc)


ROWS_BLK = 2000


def _dense_body(h_ref, ah_ref, nrm_ref, w1_ref, w2_ref, b_ref, g_ref, lb_ref,
                o_ref):
    x = jnp.dot(h_ref[...], w1_ref[...], preferred_element_type=jnp.float32)
    ahn = ah_ref[...] * nrm_ref[...]
    x = x + jnp.dot(ahn, w2_ref[...], preferred_element_type=jnp.float32)
    x = x + b_ref[...]
    mean = jnp.mean(x, axis=1, keepdims=True)
    xc = x - mean
    var = jnp.mean(xc * xc, axis=1, keepdims=True)
    y = xc * lax.rsqrt(var + 1e-5) * g_ref[...] + lb_ref[...]
    o_ref[...] = jnp.maximum(y, 0.0)


def _dense_tc(h, ah, norm, w1t, w2t, b2, g2, lb2):
    grid = (N_NODES // ROWS_BLK,)
    return pl.pallas_call(
        _dense_body,
        grid=grid,
        in_specs=[
            pl.BlockSpec((ROWS_BLK, IN_FEATS), lambda i: (i, 0)),
            pl.BlockSpec((ROWS_BLK, F_EDGE), lambda i: (i, 0)),
            pl.BlockSpec((ROWS_BLK, 1), lambda i: (i, 0)),
            pl.BlockSpec((IN_FEATS, OUT_FEATS), lambda i: (0, 0)),
            pl.BlockSpec((F_EDGE, OUT_FEATS), lambda i: (0, 0)),
            pl.BlockSpec((1, OUT_FEATS), lambda i: (0, 0)),
            pl.BlockSpec((1, OUT_FEATS), lambda i: (0, 0)),
            pl.BlockSpec((1, OUT_FEATS), lambda i: (0, 0)),
        ],
        out_specs=pl.BlockSpec((ROWS_BLK, OUT_FEATS), lambda i: (i, 0)),
        out_shape=jax.ShapeDtypeStruct((N_NODES, OUT_FEATS), jnp.float32),
    )(h, ah, norm, w1t, w2t, b2, g2, lb2)


def kernel(h, edge_m, norm, edge_index, W, b, ln_g, ln_b):
    dst = edge_index[1].astype(jnp.int32)
    ah = _segment_sum_sc(edge_m, dst)
    w1t = W[:, :IN_FEATS].T
    w2t = W[:, IN_FEATS:].T
    return _dense_tc(h, ah, norm, w1t, w2t,
                     b.reshape(1, -1), ln_g.reshape(1, -1),
                     ln_b.reshape(1, -1))
